# baseline (speedup 1.0000x reference)
import functools

import jax
import jax.numpy as jnp
from jax import lax
from jax.experimental import pallas as pl
from jax.experimental.pallas import tpu as pltpu
from jax.experimental.pallas import tpu_sc as plsc

T = 2048
C = 768
H = 12
HD = 64
NMEM = 8192
TOPK = 3
TB = 512
NT = T // TB

def _qkv_body(x_ref, w_ref, q_ref, kv_ref):
    qkv = jnp.dot(x_ref[...], w_ref[...], preferred_element_type=jnp.float32)
    q_ref[...] = qkv[:, 0:C]
    kv_ref[...] = qkv[:, C:3 * C]


def _qkv_call(x2, c_attn_w):
    return pl.pallas_call(
        _qkv_body,
        grid=(NT,),
        in_specs=[
            pl.BlockSpec((TB, C), lambda t: (t, 0)),
            pl.BlockSpec((C, 3 * C), lambda t: (0, 0)),
        ],
        out_specs=[
            pl.BlockSpec((TB, C), lambda t: (t, 0)),
            pl.BlockSpec((TB, 2 * C), lambda t: (t, 0)),
        ],
        out_shape=[
            jax.ShapeDtypeStruct((T, C), jnp.float32),
            jax.ShapeDtypeStruct((T, 2 * C), jnp.float32),
        ],
    )(x2, c_attn_w)


TBB = 1024


def _attn_part(q2, kv2, rblk, kext):
    def body(q_ref, k_ref, v_ref, y_ref):
        row = rblk * TBB + lax.broadcasted_iota(jnp.int32, (TBB, kext), 0)
        col = lax.broadcasted_iota(jnp.int32, (TBB, kext), 1)
        mask = col <= row
        for h in range(2):
            q = q_ref[:, h * HD:(h + 1) * HD]
            k = k_ref[:, h * HD:(h + 1) * HD]
            v = v_ref[:, h * HD:(h + 1) * HD]
            s = lax.dot_general(q, k, (((1,), (1,)), ((), ())),
                                preferred_element_type=jnp.float32)
            s = s * 0.125
            s = jnp.where(mask, s, jnp.float32(-1e30))
            m = jnp.max(s, axis=1, keepdims=True)
            e = jnp.exp(s - m)
            den = jnp.sum(e, axis=1, keepdims=True)
            y = jnp.dot(e, v, preferred_element_type=jnp.float32) / den
            y_ref[:, h * HD:(h + 1) * HD] = y

    return pl.pallas_call(
        body,
        grid=(H // 2,),
        in_specs=[
            pl.BlockSpec((TBB, 128), lambda hp: (rblk, hp)),
            pl.BlockSpec((kext, 128), lambda hp: (0, hp)),
            pl.BlockSpec((kext, 128), lambda hp: (0, 6 + hp)),
        ],
        out_specs=pl.BlockSpec((TBB, 128), lambda hp: (0, hp)),
        out_shape=jax.ShapeDtypeStruct((TBB, C), jnp.float32),
    )(q2, kv2, kv2)


def _topk_body(q_ref, mk_ref, idx_ref):
    s = lax.dot_general(q_ref[...], mk_ref[...], (((1,), (1,)), ((), ())),
                        preferred_element_type=jnp.float32)
    col = lax.broadcasted_iota(jnp.int32, (TB, NMEM), 1)
    picks = []
    for _ in range(TOPK):
        i = jnp.argmax(s, axis=1, keepdims=True).astype(jnp.int32)
        picks.append(i)
        s = jnp.where(col == i, jnp.float32(-3e38), s)
    lane = lax.broadcasted_iota(jnp.int32, (TB, 128), 1)
    out = jnp.where(lane == 0, picks[0],
                    jnp.where(lane == 1, picks[1],
                              jnp.where(lane == 2, picks[2], 0)))
    idx_ref[...] = out


def _topk_call(qkv, table):
    return pl.pallas_call(
        _topk_body,
        grid=(NT,),
        in_specs=[
            pl.BlockSpec((TB, C), lambda t: (t, 0)),
            pl.BlockSpec((NMEM, C), lambda t: (0, 0)),
        ],
        out_specs=pl.BlockSpec((TB, 128), lambda t: (t, 0)),
        out_shape=jax.ShapeDtypeStruct((T, 128), jnp.int32),
    )(qkv, table)


_NROWS = TOPK * T
_NW = 32
_RPW = _NROWS // _NW
_CHUNK = 48


def _gather_call(table, idx_flat):
    mesh = plsc.VectorSubcoreMesh(core_axis_name="c", subcore_axis_name="s")

    @functools.partial(
        pl.kernel,
        mesh=mesh,
        out_type=jax.ShapeDtypeStruct((_NROWS, 2 * C), jnp.float32),
        scratch_types=[
            pltpu.VMEM((_RPW,), jnp.int32),
            pltpu.VMEM((_CHUNK, 2 * C), jnp.float32),
            pltpu.SemaphoreType.DMA,
        ],
    )
    def _gather(table_hbm, idx_hbm, out_hbm, idx_v, rows_v, sem):
        wid = lax.axis_index("s") * 2 + lax.axis_index("c")
        base = wid * _RPW
        pltpu.sync_copy(idx_hbm.at[pl.ds(base, _RPW)], idx_v)
        for ch in range(_RPW // _CHUNK):
            pltpu.async_copy(
                table_hbm.at[idx_v.at[pl.ds(ch * _CHUNK, _CHUNK)]], rows_v, sem
            ).wait()
            pltpu.sync_copy(rows_v, out_hbm.at[pl.ds(base + ch * _CHUNK, _CHUNK)])

    return _gather(table, idx_flat)


def _mem_body(q_ref, g_ref, y_ref, gate_ref, jmat_ref, w_ref, out_ref):
    q = q_ref[...]
    qk = []
    for kk in range(TOPK):
        p = q * g_ref[kk][:, 0:C]
        qk.append(jnp.dot(p, jmat_ref[...], preferred_element_type=jnp.float32)
                  * 0.125)
    m = jnp.maximum(jnp.maximum(qk[0], qk[1]), qk[2])
    e = [jnp.exp(x - m) for x in qk]
    den = e[0] + e[1] + e[2]
    mem = (e[0] * g_ref[0][:, C:2 * C]
           + e[1] * g_ref[1][:, C:2 * C]
           + e[2] * g_ref[2][:, C:2 * C]) / den
    gate = gate_ref[...]
    comb = mem * gate + y_ref[...] * (1.0 - gate)
    out_ref[...] = jnp.dot(comb, w_ref[...], preferred_element_type=jnp.float32)


def _mem_call(qkv, g, y, gate_full, jmat, c_proj_w):
    return pl.pallas_call(
        _mem_body,
        grid=(NT,),
        in_specs=[
            pl.BlockSpec((TB, C), lambda t: (t, 0)),
            pl.BlockSpec((TOPK, TB, 2 * C), lambda t: (0, t, 0)),
            pl.BlockSpec((TB, C), lambda t: (t, 0)),
            pl.BlockSpec((1, C), lambda t: (0, 0)),
            pl.BlockSpec((C, C), lambda t: (0, 0)),
            pl.BlockSpec((C, C), lambda t: (0, 0)),
        ],
        out_specs=pl.BlockSpec((TB, C), lambda t: (t, 0)),
        out_shape=jax.ShapeDtypeStruct((T, C), jnp.float32),
    )(qkv, g, y, gate_full, jmat, c_proj_w)


def kernel(x, db_kv, c_attn_w, c_proj_w, gate_bias):
    x2 = x[0]
    db3 = db_kv[0]

    q2, kv2 = _qkv_call(x2, c_attn_w)
    table = db3.reshape(NMEM, 2 * C)
    y0 = _attn_part(q2, kv2, 0, T // 2)
    idxpad = _topk_call(q2, table)
    idx_flat = idxpad[:, :TOPK].T.reshape(-1)

    g = _gather_call(table, idx_flat).reshape(TOPK, T, 2 * C)
    y1 = _attn_part(q2, kv2, 1, T)
    y = jnp.concatenate([y0, y1], axis=0)

    gate_full = jnp.repeat(gate_bias.reshape(H), HD)[None, :]
    seg = jnp.arange(C, dtype=jnp.int32) // HD
    jmat = (seg[:, None] == seg[None, :]).astype(jnp.float32)

    out = _mem_call(q2, g, y, gate_full, jmat, c_proj_w)
    return out[None], kv2.reshape(T, 2, C)[None]


# speedup vs baseline: 1.1113x; 1.1113x over previous
import functools

import jax
import jax.numpy as jnp
from jax import lax
from jax.experimental import pallas as pl
from jax.experimental.pallas import tpu as pltpu
from jax.experimental.pallas import tpu_sc as plsc

T = 2048
C = 768
H = 12
HD = 64
NMEM = 8192
TOPK = 3
TB = 512
NT = T // TB

def _qkv_body(x_ref, w_ref, q_ref, kv_ref, kvm_ref):
    qkv = jnp.dot(x_ref[...], w_ref[...], preferred_element_type=jnp.float32)
    q_ref[...] = qkv[:, 0:C]
    kv = qkv[:, C:3 * C]
    kv_ref[...] = kv
    kvm_ref[:, 0, :] = kv[:, 0:C]
    kvm_ref[:, 1, :] = kv[:, C:2 * C]


def _qkv_call(x2, c_attn_w):
    return pl.pallas_call(
        _qkv_body,
        grid=(NT,),
        in_specs=[
            pl.BlockSpec((TB, C), lambda t: (t, 0)),
            pl.BlockSpec((C, 3 * C), lambda t: (0, 0)),
        ],
        out_specs=[
            pl.BlockSpec((TB, C), lambda t: (t, 0)),
            pl.BlockSpec((TB, 2 * C), lambda t: (t, 0)),
            pl.BlockSpec((TB, 2, C), lambda t: (t, 0, 0)),
        ],
        out_shape=[
            jax.ShapeDtypeStruct((T, C), jnp.float32),
            jax.ShapeDtypeStruct((T, 2 * C), jnp.float32),
            jax.ShapeDtypeStruct((T, 2, C), jnp.float32),
        ],
    )(x2, c_attn_w)


TBB = 1024


def _attn_part(q2, kv2, rblk, kext):
    def body(q_ref, k_ref, v_ref, y_ref):
        row = rblk * TBB + lax.broadcasted_iota(jnp.int32, (TBB, kext), 0)
        col = lax.broadcasted_iota(jnp.int32, (TBB, kext), 1)
        mask = col <= row
        for h in range(2):
            q = q_ref[:, h * HD:(h + 1) * HD]
            k = k_ref[:, h * HD:(h + 1) * HD]
            v = v_ref[:, h * HD:(h + 1) * HD]
            s = lax.dot_general(q, k, (((1,), (1,)), ((), ())),
                                preferred_element_type=jnp.float32)
            s = s * 0.125
            s = jnp.where(mask, s, jnp.float32(-1e30))
            m = jnp.max(s, axis=1, keepdims=True)
            e = jnp.exp(s - m)
            den = jnp.sum(e, axis=1, keepdims=True)
            y = jnp.dot(e, v, preferred_element_type=jnp.float32) / den
            y_ref[:, h * HD:(h + 1) * HD] = y

    return pl.pallas_call(
        body,
        grid=(H // 2,),
        in_specs=[
            pl.BlockSpec((TBB, 128), lambda hp: (rblk, hp)),
            pl.BlockSpec((kext, 128), lambda hp: (0, hp)),
            pl.BlockSpec((kext, 128), lambda hp: (0, 6 + hp)),
        ],
        out_specs=pl.BlockSpec((TBB, 128), lambda hp: (0, hp)),
        out_shape=jax.ShapeDtypeStruct((TBB, C), jnp.float32),
    )(q2, kv2, kv2)


def _topk_body(q_ref, mk_ref, idx_ref):
    s = lax.dot_general(q_ref[...], mk_ref[...], (((1,), (1,)), ((), ())),
                        preferred_element_type=jnp.float32)
    col = lax.broadcasted_iota(jnp.int32, (TB, NMEM), 1)
    picks = []
    for _ in range(TOPK):
        i = jnp.argmax(s, axis=1, keepdims=True).astype(jnp.int32)
        picks.append(i)
        s = jnp.where(col == i, jnp.float32(-3e38), s)
    lane = lax.broadcasted_iota(jnp.int32, (TB, 128), 1)
    out = jnp.where(lane == 0, picks[0],
                    jnp.where(lane == 1, picks[1],
                              jnp.where(lane == 2, picks[2], 0)))
    idx_ref[...] = out


def _topk_call(qkv, table):
    return pl.pallas_call(
        _topk_body,
        grid=(NT,),
        in_specs=[
            pl.BlockSpec((TB, C), lambda t: (t, 0)),
            pl.BlockSpec((NMEM, C), lambda t: (0, 0)),
        ],
        out_specs=pl.BlockSpec((TB, 128), lambda t: (t, 0)),
        out_shape=jax.ShapeDtypeStruct((T, 128), jnp.int32),
    )(qkv, table)


_NROWS = TOPK * T
_NW = 32
_RPW = _NROWS // _NW
_CHUNK = 48


def _gather_call(table, idx_flat):
    mesh = plsc.VectorSubcoreMesh(core_axis_name="c", subcore_axis_name="s")

    @functools.partial(
        pl.kernel,
        mesh=mesh,
        out_type=jax.ShapeDtypeStruct((_NROWS, 2 * C), jnp.float32),
        scratch_types=[
            pltpu.VMEM((_RPW,), jnp.int32),
            pltpu.VMEM((_CHUNK, 2 * C), jnp.float32),
            pltpu.SemaphoreType.DMA,
        ],
    )
    def _gather(table_hbm, idx_hbm, out_hbm, idx_v, rows_v, sem):
        wid = lax.axis_index("s") * 2 + lax.axis_index("c")
        base = wid * _RPW
        pltpu.sync_copy(idx_hbm.at[pl.ds(base, _RPW)], idx_v)
        for ch in range(_RPW // _CHUNK):
            pltpu.async_copy(
                table_hbm.at[idx_v.at[pl.ds(ch * _CHUNK, _CHUNK)]], rows_v, sem
            ).wait()
            pltpu.sync_copy(rows_v, out_hbm.at[pl.ds(base + ch * _CHUNK, _CHUNK)])

    return _gather(table, idx_flat)


def _mem_body(q_ref, g_ref, y_ref, gate_ref, jmat_ref, w_ref, out_ref):
    q = q_ref[...]
    qk = []
    for kk in range(TOPK):
        p = q * g_ref[kk][:, 0:C]
        qk.append(jnp.dot(p, jmat_ref[...], preferred_element_type=jnp.float32)
                  * 0.125)
    m = jnp.maximum(jnp.maximum(qk[0], qk[1]), qk[2])
    e = [jnp.exp(x - m) for x in qk]
    den = e[0] + e[1] + e[2]
    mem = (e[0] * g_ref[0][:, C:2 * C]
           + e[1] * g_ref[1][:, C:2 * C]
           + e[2] * g_ref[2][:, C:2 * C]) / den
    gate = gate_ref[...]
    comb = mem * gate + y_ref[...] * (1.0 - gate)
    out_ref[...] = jnp.dot(comb, w_ref[...], preferred_element_type=jnp.float32)


def _mem_call(qkv, g, y, gate_full, jmat, c_proj_w):
    return pl.pallas_call(
        _mem_body,
        grid=(NT,),
        in_specs=[
            pl.BlockSpec((TB, C), lambda t: (t, 0)),
            pl.BlockSpec((TOPK, TB, 2 * C), lambda t: (0, t, 0)),
            pl.BlockSpec((TB, C), lambda t: (t, 0)),
            pl.BlockSpec((1, C), lambda t: (0, 0)),
            pl.BlockSpec((C, C), lambda t: (0, 0)),
            pl.BlockSpec((C, C), lambda t: (0, 0)),
        ],
        out_specs=pl.BlockSpec((TB, C), lambda t: (t, 0)),
        out_shape=jax.ShapeDtypeStruct((T, C), jnp.float32),
    )(qkv, g, y, gate_full, jmat, c_proj_w)


def kernel(x, db_kv, c_attn_w, c_proj_w, gate_bias):
    x2 = x[0]
    db3 = db_kv[0]

    q2, kv2, kvmem = _qkv_call(x2, c_attn_w)
    table = db3.reshape(NMEM, 2 * C)
    y0 = _attn_part(q2, kv2, 0, T // 2)
    idxpad = _topk_call(q2, table)
    idx_flat = idxpad[:, :TOPK].T.reshape(-1)

    g = _gather_call(table, idx_flat).reshape(TOPK, T, 2 * C)
    y1 = _attn_part(q2, kv2, 1, T)
    y = jnp.concatenate([y0, y1], axis=0)

    gate_full = jnp.repeat(gate_bias.reshape(H), HD)[None, :]
    seg = jnp.arange(C, dtype=jnp.int32) // HD
    jmat = (seg[:, None] == seg[None, :]).astype(jnp.float32)

    out = _mem_call(q2, g, y, gate_full, jmat, c_proj_w)
    return out[None], kvmem[None]


# speedup vs baseline: 1.1247x; 1.0120x over previous
import functools

import jax
import jax.numpy as jnp
from jax import lax
from jax.experimental import pallas as pl
from jax.experimental.pallas import tpu as pltpu
from jax.experimental.pallas import tpu_sc as plsc

T = 2048
C = 768
H = 12
HD = 64
NMEM = 8192
TOPK = 3
TB = 512
NT = T // TB

def _qkv_body(x_ref, w_ref, q_ref, kv_ref, kvm_ref):
    qkv = jnp.dot(x_ref[...], w_ref[...], preferred_element_type=jnp.float32)
    q_ref[...] = qkv[:, 0:C]
    kv = qkv[:, C:3 * C]
    kv_ref[...] = kv
    kvm_ref[:, 0, :] = kv[:, 0:C]
    kvm_ref[:, 1, :] = kv[:, C:2 * C]


def _qkv_call(x2, c_attn_w):
    return pl.pallas_call(
        _qkv_body,
        grid=(NT,),
        in_specs=[
            pl.BlockSpec((TB, C), lambda t: (t, 0)),
            pl.BlockSpec((C, 3 * C), lambda t: (0, 0)),
        ],
        out_specs=[
            pl.BlockSpec((TB, C), lambda t: (t, 0)),
            pl.BlockSpec((TB, 2 * C), lambda t: (t, 0)),
            pl.BlockSpec((TB, 2, C), lambda t: (t, 0, 0)),
        ],
        out_shape=[
            jax.ShapeDtypeStruct((T, C), jnp.float32),
            jax.ShapeDtypeStruct((T, 2 * C), jnp.float32),
            jax.ShapeDtypeStruct((T, 2, C), jnp.float32),
        ],
    )(x2, c_attn_w)


TBB = 1024


def _attn_part(q2, kv2, rblk, kext):
    def body(q_ref, k_ref, v_ref, y_ref):
        row = rblk * TBB + lax.broadcasted_iota(jnp.int32, (TBB, kext), 0)
        col = lax.broadcasted_iota(jnp.int32, (TBB, kext), 1)
        mask = col <= row
        for h in range(2):
            q = q_ref[:, h * HD:(h + 1) * HD]
            k = k_ref[:, h * HD:(h + 1) * HD]
            v = v_ref[:, h * HD:(h + 1) * HD]
            s = lax.dot_general(q, k, (((1,), (1,)), ((), ())),
                                preferred_element_type=jnp.float32)
            s = s * 0.125
            s = jnp.where(mask, s, jnp.float32(-1e30))
            m = jnp.max(s, axis=1, keepdims=True)
            e = jnp.exp(s - m)
            den = jnp.sum(e, axis=1, keepdims=True)
            y = jnp.dot(e, v, preferred_element_type=jnp.float32) / den
            y_ref[:, h * HD:(h + 1) * HD] = y

    return pl.pallas_call(
        body,
        grid=(H // 2,),
        in_specs=[
            pl.BlockSpec((TBB, 128), lambda hp: (rblk, hp)),
            pl.BlockSpec((kext, 128), lambda hp: (0, hp)),
            pl.BlockSpec((kext, 128), lambda hp: (0, 6 + hp)),
        ],
        out_specs=pl.BlockSpec((TBB, 128), lambda hp: (0, hp)),
        out_shape=jax.ShapeDtypeStruct((TBB, C), jnp.float32),
    )(q2, kv2, kv2)


def _topk_body(q_ref, mk_ref, idx_ref):
    s = lax.dot_general(q_ref[...], mk_ref[...], (((1,), (1,)), ((), ())),
                        preferred_element_type=jnp.float32)
    col = lax.broadcasted_iota(jnp.int32, (TB, NMEM), 1)
    picks = []
    for _ in range(TOPK):
        i = jnp.argmax(s, axis=1, keepdims=True).astype(jnp.int32)
        picks.append(i)
        s = jnp.where(col == i, jnp.float32(-3e38), s)
    lane = lax.broadcasted_iota(jnp.int32, (TB, 128), 1)
    out = jnp.where(lane == 0, picks[0],
                    jnp.where(lane == 1, picks[1],
                              jnp.where(lane == 2, picks[2], 0)))
    idx_ref[...] = out


def _topk_call(qkv, table):
    return pl.pallas_call(
        _topk_body,
        grid=(NT,),
        in_specs=[
            pl.BlockSpec((TB, C), lambda t: (t, 0)),
            pl.BlockSpec((NMEM, C), lambda t: (0, 0)),
        ],
        out_specs=pl.BlockSpec((TB, 128), lambda t: (t, 0)),
        out_shape=jax.ShapeDtypeStruct((T, 128), jnp.int32),
    )(qkv, table)


_NROWS = TOPK * T
_NW = 32
_RPW = _NROWS // _NW
_CHUNK = 48


def _gather_call(table, idx_flat):
    mesh = plsc.VectorSubcoreMesh(core_axis_name="c", subcore_axis_name="s")

    @functools.partial(
        pl.kernel,
        mesh=mesh,
        out_type=jax.ShapeDtypeStruct((_NROWS, 2 * C), jnp.float32),
        scratch_types=[
            pltpu.VMEM((_RPW,), jnp.int32),
            pltpu.VMEM((_CHUNK, 2 * C), jnp.float32),
            pltpu.SemaphoreType.DMA,
        ],
    )
    def _gather(table_hbm, idx_hbm, out_hbm, idx_v, rows_v, sem):
        wid = lax.axis_index("s") * 2 + lax.axis_index("c")
        base = wid * _RPW
        pltpu.sync_copy(idx_hbm.at[pl.ds(base, _RPW)], idx_v)
        for ch in range(_RPW // _CHUNK):
            pltpu.async_copy(
                table_hbm.at[idx_v.at[pl.ds(ch * _CHUNK, _CHUNK)]], rows_v, sem
            ).wait()
            pltpu.sync_copy(rows_v, out_hbm.at[pl.ds(base + ch * _CHUNK, _CHUNK)])

    return _gather(table, idx_flat)


def _mem_body(q_ref, g_ref, y_ref, gate_ref, sdn_ref, sup_ref, w_ref, out_ref):
    q = q_ref[...]
    qk = []
    for kk in range(TOPK):
        p = q * g_ref[kk][:, 0:C]
        qk.append(jnp.dot(p, sdn_ref[...], preferred_element_type=jnp.float32)
                  * 0.125)
    m = jnp.maximum(jnp.maximum(qk[0], qk[1]), qk[2])
    e = [jnp.exp(x - m) for x in qk]
    den = e[0] + e[1] + e[2]
    w = [jnp.dot(ek / den, sup_ref[...], preferred_element_type=jnp.float32)
         for ek in e]
    mem = (w[0] * g_ref[0][:, C:2 * C]
           + w[1] * g_ref[1][:, C:2 * C]
           + w[2] * g_ref[2][:, C:2 * C])
    gate = gate_ref[...]
    comb = mem * gate + y_ref[...] * (1.0 - gate)
    out_ref[...] = jnp.dot(comb, w_ref[...], preferred_element_type=jnp.float32)


def _mem_call(qkv, g, y, gate_full, sdn, sup, c_proj_w):
    return pl.pallas_call(
        _mem_body,
        grid=(NT,),
        in_specs=[
            pl.BlockSpec((TB, C), lambda t: (t, 0)),
            pl.BlockSpec((TOPK, TB, 2 * C), lambda t: (0, t, 0)),
            pl.BlockSpec((TB, C), lambda t: (t, 0)),
            pl.BlockSpec((1, C), lambda t: (0, 0)),
            pl.BlockSpec((C, 128), lambda t: (0, 0)),
            pl.BlockSpec((128, C), lambda t: (0, 0)),
            pl.BlockSpec((C, C), lambda t: (0, 0)),
        ],
        out_specs=pl.BlockSpec((TB, C), lambda t: (t, 0)),
        out_shape=jax.ShapeDtypeStruct((T, C), jnp.float32),
    )(qkv, g, y, gate_full, sdn, sup, c_proj_w)


def kernel(x, db_kv, c_attn_w, c_proj_w, gate_bias):
    x2 = x[0]
    db3 = db_kv[0]

    q2, kv2, kvmem = _qkv_call(x2, c_attn_w)
    table = db3.reshape(NMEM, 2 * C)
    y0 = _attn_part(q2, kv2, 0, T // 2)
    idxpad = _topk_call(q2, table)
    idx_flat = idxpad[:, :TOPK].T.reshape(-1)

    g = _gather_call(table, idx_flat).reshape(TOPK, T, 2 * C)
    y1 = _attn_part(q2, kv2, 1, T)
    y = jnp.concatenate([y0, y1], axis=0)

    gate_full = jnp.repeat(gate_bias.reshape(H), HD)[None, :]
    seg = jnp.arange(C, dtype=jnp.int32) // HD
    lane = jnp.arange(128, dtype=jnp.int32)
    sdn = (seg[:, None] == lane[None, :]).astype(jnp.float32)
    sup = (lane[:, None] == seg[None, :]).astype(jnp.float32)

    out = _mem_call(q2, g, y, gate_full, sdn, sup, c_proj_w)
    return out[None], kvmem[None]


# speedup vs baseline: 1.1897x; 1.0578x over previous
import functools

import jax
import jax.numpy as jnp
from jax import lax
from jax.experimental import pallas as pl
from jax.experimental.pallas import tpu as pltpu
from jax.experimental.pallas import tpu_sc as plsc

T = 2048
C = 768
H = 12
HD = 64
NMEM = 8192
TOPK = 3
TB = 512
NT = T // TB

def _qkv_body(x_ref, w_ref, q_ref, kv_ref, kvm_ref):
    qkv = jnp.dot(x_ref[...], w_ref[...], preferred_element_type=jnp.float32)
    q_ref[...] = qkv[:, 0:C]
    kv = qkv[:, C:3 * C]
    kv_ref[...] = kv
    kvm_ref[:, 0, :] = kv[:, 0:C]
    kvm_ref[:, 1, :] = kv[:, C:2 * C]


def _qkv_call(x2, c_attn_w):
    return pl.pallas_call(
        _qkv_body,
        grid=(NT,),
        in_specs=[
            pl.BlockSpec((TB, C), lambda t: (t, 0)),
            pl.BlockSpec((C, 3 * C), lambda t: (0, 0)),
        ],
        out_specs=[
            pl.BlockSpec((TB, C), lambda t: (t, 0)),
            pl.BlockSpec((TB, 2 * C), lambda t: (t, 0)),
            pl.BlockSpec((TB, 2, C), lambda t: (t, 0, 0)),
        ],
        out_shape=[
            jax.ShapeDtypeStruct((T, C), jnp.float32),
            jax.ShapeDtypeStruct((T, 2 * C), jnp.float32),
            jax.ShapeDtypeStruct((T, 2, C), jnp.float32),
        ],
    )(x2, c_attn_w)


TBB = 1024


def _attn_part(q2, kv2, rblk, kext):
    def body(q_ref, k_ref, v_ref, y_ref):
        row = rblk * TBB + lax.broadcasted_iota(jnp.int32, (TBB, kext), 0)
        col = lax.broadcasted_iota(jnp.int32, (TBB, kext), 1)
        mask = col <= row
        for h in range(2):
            q = q_ref[:, h * HD:(h + 1) * HD]
            k = k_ref[:, h * HD:(h + 1) * HD]
            v = v_ref[:, h * HD:(h + 1) * HD]
            s = lax.dot_general(q, k, (((1,), (1,)), ((), ())),
                                preferred_element_type=jnp.float32)
            s = s * 0.125
            s = jnp.where(mask, s, jnp.float32(-1e30))
            m = jnp.max(s, axis=1, keepdims=True)
            e = jnp.exp(s - m)
            den = jnp.sum(e, axis=1, keepdims=True)
            y = jnp.dot(e, v, preferred_element_type=jnp.float32) / den
            y_ref[:, h * HD:(h + 1) * HD] = y

    return pl.pallas_call(
        body,
        grid=(H // 2,),
        in_specs=[
            pl.BlockSpec((TBB, 128), lambda hp: (rblk, hp)),
            pl.BlockSpec((kext, 128), lambda hp: (0, hp)),
            pl.BlockSpec((kext, 128), lambda hp: (0, 6 + hp)),
        ],
        out_specs=pl.BlockSpec((TBB, 128), lambda hp: (0, hp)),
        out_shape=jax.ShapeDtypeStruct((TBB, C), jnp.float32),
    )(q2, kv2, kv2)


def _topk_body(q_ref, mk_ref, idx_ref):
    s = lax.dot_general(q_ref[...], mk_ref[...], (((1,), (1,)), ((), ())),
                        preferred_element_type=jnp.float32)
    col = lax.broadcasted_iota(jnp.int32, (TB, NMEM), 1)
    picks = []
    for _ in range(TOPK):
        i = jnp.argmax(s, axis=1, keepdims=True).astype(jnp.int32)
        picks.append(i)
        s = jnp.where(col == i, jnp.float32(-3e38), s)
    lane = lax.broadcasted_iota(jnp.int32, (TB, 128), 1)
    out = jnp.where(lane == 0, picks[0],
                    jnp.where(lane == 1, picks[1],
                              jnp.where(lane == 2, picks[2], 0)))
    idx_ref[...] = out


def _topk_call(qkv, table):
    return pl.pallas_call(
        _topk_body,
        grid=(NT,),
        in_specs=[
            pl.BlockSpec((TB, C), lambda t: (t, 0)),
            pl.BlockSpec((NMEM, C), lambda t: (0, 0)),
        ],
        out_specs=pl.BlockSpec((TB, 128), lambda t: (t, 0)),
        out_shape=jax.ShapeDtypeStruct((T, 128), jnp.int32),
    )(qkv, table)


_NROWS = TOPK * T
_NW = 32
_RPW = _NROWS // _NW
_CHUNK = 48


def _gather_call(table, idx_flat):
    mesh = plsc.VectorSubcoreMesh(core_axis_name="c", subcore_axis_name="s")

    @functools.partial(
        pl.kernel,
        mesh=mesh,
        out_type=jax.ShapeDtypeStruct((_NROWS, 2 * C), jnp.float32),
        scratch_types=[
            pltpu.VMEM((_RPW,), jnp.int32),
            pltpu.VMEM((_CHUNK, 2 * C), jnp.float32),
            pltpu.SemaphoreType.DMA,
        ],
    )
    def _gather(table_hbm, idx_hbm, out_hbm, idx_v, rows_v, sem):
        wid = lax.axis_index("s") * 2 + lax.axis_index("c")
        base = wid * _RPW
        pltpu.sync_copy(idx_hbm.at[pl.ds(base, _RPW)], idx_v)
        for ch in range(_RPW // _CHUNK):
            pltpu.async_copy(
                table_hbm.at[idx_v.at[pl.ds(ch * _CHUNK, _CHUNK)]], rows_v, sem
            ).wait()
            pltpu.sync_copy(rows_v, out_hbm.at[pl.ds(base + ch * _CHUNK, _CHUNK)])

    return _gather(table, idx_flat)


def _mem_body(q_ref, g_ref, y_ref, gate_ref, sdn_ref, sup_ref, w_ref, out_ref):
    q = q_ref[...]
    qk = []
    for kk in range(TOPK):
        p = q * g_ref[kk][:, 0:C]
        qk.append(jnp.dot(p, sdn_ref[...], preferred_element_type=jnp.float32)
                  * 0.125)
    m = jnp.maximum(jnp.maximum(qk[0], qk[1]), qk[2])
    e = [jnp.exp(x - m) for x in qk]
    den = e[0] + e[1] + e[2]
    w = [jnp.dot(ek / den, sup_ref[...], preferred_element_type=jnp.float32)
         for ek in e]
    mem = (w[0] * g_ref[0][:, C:2 * C]
           + w[1] * g_ref[1][:, C:2 * C]
           + w[2] * g_ref[2][:, C:2 * C])
    gate = gate_ref[...]
    comb = mem * gate + y_ref[...] * (1.0 - gate)
    out_ref[...] = jnp.dot(comb, w_ref[...], preferred_element_type=jnp.float32)


def _mem_call(qkv, g, y, gate_full, sdn, sup, c_proj_w):
    return pl.pallas_call(
        _mem_body,
        grid=(NT,),
        in_specs=[
            pl.BlockSpec((TB, C), lambda t: (t, 0)),
            pl.BlockSpec((TOPK, TB, 2 * C), lambda t: (0, t, 0)),
            pl.BlockSpec((TB, C), lambda t: (t, 0)),
            pl.BlockSpec((1, C), lambda t: (0, 0)),
            pl.BlockSpec((C, 128), lambda t: (0, 0)),
            pl.BlockSpec((128, C), lambda t: (0, 0)),
            pl.BlockSpec((C, C), lambda t: (0, 0)),
        ],
        out_specs=pl.BlockSpec((TB, C), lambda t: (t, 0)),
        out_shape=jax.ShapeDtypeStruct((T, C), jnp.float32),
    )(qkv, g, y, gate_full, sdn, sup, c_proj_w)


def kernel(x, db_kv, c_attn_w, c_proj_w, gate_bias):
    x2 = x[0]
    db3 = db_kv[0]

    q2, kv2, kvmem = _qkv_call(x2, c_attn_w)
    table = db3.reshape(NMEM, 2 * C)
    y0 = _attn_part(q2, kv2, 0, T // 2)
    table, y0 = lax.optimization_barrier((table, y0))
    idxpad = _topk_call(q2, table)
    idx_flat = idxpad[:, :TOPK].T.reshape(-1)

    g = _gather_call(table, idx_flat).reshape(TOPK, T, 2 * C)
    y1 = _attn_part(q2, kv2, 1, T)
    y = jnp.concatenate([y0, y1], axis=0)

    gate_full = jnp.repeat(gate_bias.reshape(H), HD)[None, :]
    seg = jnp.arange(C, dtype=jnp.int32) // HD
    lane = jnp.arange(128, dtype=jnp.int32)
    sdn = (seg[:, None] == lane[None, :]).astype(jnp.float32)
    sup = (lane[:, None] == seg[None, :]).astype(jnp.float32)

    out = _mem_call(q2, g, y, gate_full, sdn, sup, c_proj_w)
    return out[None], kvmem[None]
